# Initial kernel scaffold; baseline (speedup 1.0000x reference)
import jax
import jax.numpy as jnp
from jax.experimental import pallas as pl


def kernel(points, params, batch_size):
    raise NotImplementedError("write your pallas kernel here")


# speedup vs baseline: 9.6194x; 9.6194x over previous
import functools

import jax
import jax.numpy as jnp
from jax import lax
from jax.experimental import pallas as pl
from jax.experimental.pallas import tpu as pltpu
from jax.experimental.pallas import tpu_sc as plsc


def _fps_body(m, n, x_ref, y_ref, z_ref, o_ref):
    x = x_ref[...]
    y = y_ref[...]
    z = z_ref[...]
    B, _, n8 = x.shape
    m8 = m // 8
    fid = (lax.broadcasted_iota(jnp.int32, (1, 8, n8), 1) * n8
           + lax.broadcasted_iota(jnp.int32, (1, 8, n8), 2))
    fido = (lax.broadcasted_iota(jnp.int32, (1, 8, m8), 1) * m8
            + lax.broadcasted_iota(jnp.int32, (1, 8, m8), 2))

    def rmax(a):
        return jnp.max(jnp.max(a, axis=2, keepdims=True), axis=1, keepdims=True)

    def rmin(a):
        return jnp.min(jnp.min(a, axis=2, keepdims=True), axis=1, keepdims=True)

    def rsum(a):
        return jnp.sum(jnp.sum(a, axis=2, keepdims=True), axis=1, keepdims=True)

    def step(i, state):
        dists, last, out = state
        sel = (fid == last).astype(x.dtype)
        lx = rsum(x * sel)
        ly = rsum(y * sel)
        lz = rsum(z * sel)
        dx = x - lx
        dy = y - ly
        dz = z - lz
        d = dx * dx + dy * dy + dz * dz
        dists = jnp.minimum(dists, d)
        mv = rmax(dists)
        nxt = rmin(jnp.where(dists == mv, fid, n))
        out = jnp.where(fido == i, nxt, out)
        return dists, nxt, out

    dists0 = jnp.full(x.shape, 1e10, x.dtype)
    last0 = jnp.zeros((B, 1, 1), jnp.int32)
    out0 = jnp.zeros((B, 8, m8), jnp.int32)
    _, _, out = lax.fori_loop(1, m, step, (dists0, last0, out0))
    o_ref[...] = out


def _fps(pos_t, m):
    B, _, n = pos_t.shape
    x8 = pos_t[:, 0].reshape(B, 8, n // 8)
    y8 = pos_t[:, 1].reshape(B, 8, n // 8)
    z8 = pos_t[:, 2].reshape(B, 8, n // 8)
    out = pl.pallas_call(
        functools.partial(_fps_body, m, n),
        out_shape=jax.ShapeDtypeStruct((B, 8, m // 8), jnp.int32),
    )(x8, y8, z8)
    return out.reshape(B, m)


def _sa_knn_body(n, k, pm_ref, q_ref, idx_ref, qp_ref):
    b = pl.program_id(0)
    pm = pm_ref[0]
    qidx = q_ref[0]
    iota_c = lax.broadcasted_iota(jnp.int32, (n, 1), 0)
    xc = pm[:, 0:1]
    yc = pm[:, 1:2]
    zc = pm[:, 2:3]
    oh = (iota_c == qidx).astype(jnp.float32)
    qx = jnp.sum(xc * oh, axis=0, keepdims=True)
    qy = jnp.sum(yc * oh, axis=0, keepdims=True)
    qz = jnp.sum(zc * oh, axis=0, keepdims=True)
    qrow = jnp.concatenate([qx, qy, qz], axis=0)
    qp_ref[0] = qrow
    rsq = xc * xc + yc * yc + zc * zc
    qsq = qx * qx + qy * qy + qz * qz
    m2 = lax.dot_general(pm.astype(jnp.bfloat16),
                         (2.0 * qrow).astype(jnp.bfloat16),
                         (((1,), (0,)), ((), ())),
                         preferred_element_type=jnp.float32)
    d2 = (qsq - m2) + rsq
    off = b * n
    big = jnp.float32(3.0e38)
    for j in range(k):
        mv = jnp.min(d2, axis=0, keepdims=True)
        selj = jnp.min(jnp.where(d2 == mv, iota_c, n), axis=0, keepdims=True)
        idx_ref[0, j:j + 1, :] = selj + off
        d2 = jnp.where(iota_c == selj, big, d2)


def _sa_knn(pos_mat, fps_idx, k, mb):
    B, n, _ = pos_mat.shape
    m = fps_idx.shape[1]
    qi = fps_idx.reshape(B, 1, m)
    grid = (B, m // mb)
    idx, qp = pl.pallas_call(
        functools.partial(_sa_knn_body, n, k),
        grid=grid,
        in_specs=[
            pl.BlockSpec((1, n, 3), lambda b, q: (b, 0, 0)),
            pl.BlockSpec((1, 1, mb), lambda b, q: (b, 0, q)),
        ],
        out_specs=[
            pl.BlockSpec((1, k, mb), lambda b, q: (b, 0, q)),
            pl.BlockSpec((1, 3, mb), lambda b, q: (b, 0, q)),
        ],
        out_shape=[
            jax.ShapeDtypeStruct((B, k, m), jnp.int32),
            jax.ShapeDtypeStruct((B, 3, m), jnp.float32),
        ],
    )(pos_mat, qi)
    return idx, qp


def _fp_knn_body(n, pm_ref, q_ref, idx_ref, w_ref):
    b = pl.program_id(0)
    pm = pm_ref[0]
    qrow = q_ref[0]
    xc = pm[:, 0:1]
    yc = pm[:, 1:2]
    zc = pm[:, 2:3]
    qx = qrow[0:1, :]
    qy = qrow[1:2, :]
    qz = qrow[2:3, :]
    rsq = xc * xc + yc * yc + zc * zc
    qsq = qx * qx + qy * qy + qz * qz
    iota_c = lax.broadcasted_iota(jnp.int32, (n, 1), 0)
    m2 = lax.dot_general(pm.astype(jnp.bfloat16),
                         (2.0 * qrow).astype(jnp.bfloat16),
                         (((1,), (0,)), ((), ())),
                         preferred_element_type=jnp.float32)
    d2 = (qsq - m2) + rsq
    off = b * n
    big = jnp.float32(3.0e38)
    for j in range(3):
        mv = jnp.min(d2, axis=0, keepdims=True)
        selj = jnp.min(jnp.where(d2 == mv, iota_c, n), axis=0, keepdims=True)
        idx_ref[0, j:j + 1, :] = selj + off
        w_ref[0, j:j + 1, :] = mv
        d2 = jnp.where(iota_c == selj, big, d2)


def _fp_knn(pos_mat_coarse, pos_t_fine, mb):
    B, mc, _ = pos_mat_coarse.shape
    mf = pos_t_fine.shape[2]
    grid = (B, mf // mb)
    idx, w = pl.pallas_call(
        functools.partial(_fp_knn_body, mc),
        grid=grid,
        in_specs=[
            pl.BlockSpec((1, mc, 3), lambda b, q: (b, 0, 0)),
            pl.BlockSpec((1, 3, mb), lambda b, q: (b, 0, q)),
        ],
        out_specs=[
            pl.BlockSpec((1, 3, mb), lambda b, q: (b, 0, q)),
            pl.BlockSpec((1, 3, mb), lambda b, q: (b, 0, q)),
        ],
        out_shape=[
            jax.ShapeDtypeStruct((B, 3, mf), jnp.int32),
            jax.ShapeDtypeStruct((B, 3, mf), jnp.float32),
        ],
    )(pos_mat_coarse, pos_t_fine)
    return idx, w


def _sc_gather(table, idx):
    V, D = table.shape
    Bn = idx.shape[0]
    info = plsc.get_sparse_core_info()
    nc = info.num_cores
    nw = nc * info.num_subcores
    b_per_w = Bn // nw
    mesh = plsc.VectorSubcoreMesh(core_axis_name="c", subcore_axis_name="s")

    @functools.partial(
        pl.kernel,
        mesh=mesh,
        compiler_params=pltpu.CompilerParams(use_tc_tiling_on_sc=False),
        out_type=jax.ShapeDtypeStruct((Bn, D), jnp.float32),
        scratch_types=[
            pltpu.VMEM((b_per_w,), jnp.int32),
            pltpu.VMEM((b_per_w, D), jnp.float32),
            pltpu.SemaphoreType.DMA,
        ],
    )
    def gather_kernel(table_hbm, idx_hbm, out_hbm, idx_v, rows_v, sem):
        wid = lax.axis_index("s") * nc + lax.axis_index("c")
        base = wid * b_per_w
        pltpu.sync_copy(idx_hbm.at[pl.ds(base, b_per_w)], idx_v)
        pltpu.async_copy(table_hbm.at[idx_v], rows_v, sem).wait()
        pltpu.sync_copy(rows_v, out_hbm.at[pl.ds(base, b_per_w)])

    return gather_kernel(table, idx)


def _gather_rows(table, idx_flat):
    r = idx_flat.shape[0]
    rpad = -(-r // 256) * 256
    if rpad != r:
        idx_flat = jnp.pad(idx_flat, (0, rpad - r))
    rows = _sc_gather(table, idx_flat)
    return rows[:r]


def _sa_mlp_body(k, D, g_ref, q_ref, w1_ref, b1_ref, w2_ref, b2_ref,
                 w3_ref, b3_ref, o_ref):
    qm = q_ref[0]
    mb = qm.shape[0]
    qpad = jnp.concatenate(
        [qm, jnp.zeros((mb, D - 3), jnp.float32)], axis=1)
    w1 = w1_ref[...].astype(jnp.bfloat16)
    b1 = b1_ref[...]
    w2 = w2_ref[...].astype(jnp.bfloat16)
    b2 = b2_ref[...]
    w3 = w3_ref[...].astype(jnp.bfloat16)
    b3 = b3_ref[...]
    acc = None
    for j in range(k):
        xj = (g_ref[0, j] - qpad).astype(jnp.bfloat16)
        h = jnp.dot(xj, w1, preferred_element_type=jnp.float32) + b1
        h = jnp.maximum(h, 0.0).astype(jnp.bfloat16)
        h = jnp.dot(h, w2, preferred_element_type=jnp.float32) + b2
        h = jnp.maximum(h, 0.0).astype(jnp.bfloat16)
        h = jnp.dot(h, w3, preferred_element_type=jnp.float32) + b3
        h = jnp.maximum(h, 0.0)
        acc = h if acc is None else jnp.maximum(acc, h)
    o_ref[0] = acc


def _sa_mlp(g, qpos_mat, ws, mb):
    B, k, m, D = g.shape
    (W1, b1), (W2, b2), (W3, b3) = ws
    h1 = W1.shape[1]
    h2 = W2.shape[1]
    h3 = W3.shape[1]
    W1cat = jnp.pad(W1, ((0, D - W1.shape[0]), (0, 0)))
    grid = (B, m // mb)
    out = pl.pallas_call(
        functools.partial(_sa_mlp_body, k, D),
        grid=grid,
        in_specs=[
            pl.BlockSpec((1, k, mb, D), lambda b, q: (b, 0, q, 0)),
            pl.BlockSpec((1, mb, 3), lambda b, q: (b, q, 0)),
            pl.BlockSpec((D, h1), lambda b, q: (0, 0)),
            pl.BlockSpec((1, h1), lambda b, q: (0, 0)),
            pl.BlockSpec((h1, h2), lambda b, q: (0, 0)),
            pl.BlockSpec((1, h2), lambda b, q: (0, 0)),
            pl.BlockSpec((h2, h3), lambda b, q: (0, 0)),
            pl.BlockSpec((1, h3), lambda b, q: (0, 0)),
        ],
        out_specs=pl.BlockSpec((1, mb, h3), lambda b, q: (b, q, 0)),
        out_shape=jax.ShapeDtypeStruct((B, m, h3), jnp.float32),
    )(g, qpos_mat, W1cat, b1.reshape(1, h1), W2, b2.reshape(1, h2),
      W3, b3.reshape(1, h3))
    return out


def _fp_mlp_body(nlayers, has_skip, *refs):
    if has_skip:
        (g_ref, w_ref, s_ref, wt_ref, wb_ref, b1_ref), rest = refs[:6], refs[6:]
    else:
        (g_ref, w_ref, wt_ref, b1_ref), rest = refs[:4], refs[4:]
    o_ref = rest[-1]
    layer_refs = rest[:-1]
    wcols = w_ref[0]
    s = None
    for j in range(3):
        gj = g_ref[0, j]
        term = gj * wcols[:, j:j + 1]
        s = term if s is None else s + term
    s = s.astype(jnp.bfloat16)
    h = jnp.dot(s, wt_ref[...].astype(jnp.bfloat16),
                preferred_element_type=jnp.float32)
    if has_skip:
        h = h + jnp.dot(s_ref[0].astype(jnp.bfloat16),
                        wb_ref[...].astype(jnp.bfloat16),
                        preferred_element_type=jnp.float32)
    h = jnp.maximum(h + b1_ref[...], 0.0)
    for i in range(nlayers - 1):
        wl = layer_refs[2 * i][...].astype(jnp.bfloat16)
        bl = layer_refs[2 * i + 1][...]
        h = jnp.dot(h.astype(jnp.bfloat16), wl,
                    preferred_element_type=jnp.float32)
        h = jnp.maximum(h + bl, 0.0)
    o_ref[0] = h


def _fp_mlp(gf, w, skip, ws, mb):
    B, _, mf, C = gf.shape
    nlayers = len(ws)
    (W1, b1) = ws[0]
    h1 = W1.shape[1]
    Wt = W1[:C]
    has_skip = skip is not None
    grid = (B, mf // mb)
    in_specs = [
        pl.BlockSpec((1, 3, mb, C), lambda b, q: (b, 0, q, 0)),
        pl.BlockSpec((1, mb, 3), lambda b, q: (b, q, 0)),
    ]
    args = [gf, w]
    if has_skip:
        Cs = skip.shape[-1]
        Wb = W1[C:C + Cs]
        in_specs.append(pl.BlockSpec((1, mb, Cs), lambda b, q: (b, q, 0)))
        args.append(skip)
        in_specs.append(pl.BlockSpec((C, h1), lambda b, q: (0, 0)))
        args.append(Wt)
        in_specs.append(pl.BlockSpec((Cs, h1), lambda b, q: (0, 0)))
        args.append(Wb)
    else:
        in_specs.append(pl.BlockSpec((C, h1), lambda b, q: (0, 0)))
        args.append(Wt)
    in_specs.append(pl.BlockSpec((1, h1), lambda b, q: (0, 0)))
    args.append(b1.reshape(1, h1))
    hL = h1
    for (Wl, bl) in ws[1:]:
        hin, hout = Wl.shape
        in_specs.append(pl.BlockSpec((hin, hout), lambda b, q: (0, 0)))
        args.append(Wl)
        in_specs.append(pl.BlockSpec((1, hout), lambda b, q: (0, 0)))
        args.append(bl.reshape(1, hout))
        hL = hout
    out = pl.pallas_call(
        functools.partial(_fp_mlp_body, nlayers, has_skip),
        grid=grid,
        in_specs=in_specs,
        out_specs=pl.BlockSpec((1, mb, hL), lambda b, q: (b, q, 0)),
        out_shape=jax.ShapeDtypeStruct((B, mf, hL), jnp.float32),
    )(*args)
    return out


def _sa_level(pos_t, feat, stride, nsample, ws, knn_mb, mlp_mb):
    B, _, n = pos_t.shape
    m = n // stride
    C = feat.shape[-1]
    D = -(-(3 + C) // 16) * 16
    pos_mat = jnp.transpose(pos_t, (0, 2, 1))
    fps_idx = _fps(pos_t, m)
    idx, qpos_t = _sa_knn(pos_mat, fps_idx, nsample, knn_mb)
    table = jnp.concatenate([pos_mat, feat], axis=-1)
    table = jnp.pad(table, ((0, 0), (0, 0), (0, D - 3 - C))).reshape(B * n, D)
    rows = _gather_rows(table, idx.reshape(-1))
    g = rows.reshape(B, nsample, m, D)
    new_feat = _sa_mlp(g, jnp.transpose(qpos_t, (0, 2, 1)), ws, mlp_mb)
    return qpos_t, new_feat


def _fp_level(pos_t_fine, skip, pos_t_coarse, feat_coarse, ws, knn_mb, mlp_mb):
    B, _, mc = pos_t_coarse.shape
    mf = pos_t_fine.shape[2]
    C = feat_coarse.shape[-1]
    pos_mat_coarse = jnp.transpose(pos_t_coarse, (0, 2, 1))
    idx, d2 = _fp_knn(pos_mat_coarse, pos_t_fine, knn_mb)
    w = 1.0 / (jnp.transpose(d2, (0, 2, 1)) + 1e-8)
    w = w / jnp.sum(w, axis=-1, keepdims=True)
    table = feat_coarse.reshape(B * mc, C)
    rows = _gather_rows(table, idx.reshape(-1))
    gf = rows.reshape(B, 3, mf, C)
    return _fp_mlp(gf, w, skip, ws, mlp_mb)


def kernel(points, params, batch_size):
    B = 2
    N = points.shape[0]
    n = N // B
    pos = points[:, 1:4].reshape(B, n, 3)
    feat = points[:, 4:].reshape(B, n, -1)
    pos_t0 = jnp.transpose(pos, (0, 2, 1))

    p1, f1 = _sa_level(pos_t0, feat, 4, 32, params['sa1'], 256, 256)
    p2, f2 = _sa_level(p1, f1, 4, 32, params['sa2'], 256, 256)
    p3, f3 = _sa_level(p2, f2, 4, 32, params['sa3'], 64, 64)
    p4, f4 = _sa_level(p3, f3, 4, 32, params['sa4'], 16, 16)

    f3 = _fp_level(p3, f3, p4, f4, params['fp4'], 64, 64)
    f2 = _fp_level(p2, f2, p3, f3, params['fp3'], 256, 256)
    f1 = _fp_level(p1, f1, p2, f2, params['fp2'], 512, 512)
    f0 = _fp_level(pos_t0, None, p1, f1, params['fp1'], 512, 512)
    return f0.reshape(N, -1)
